# Initial kernel scaffold; baseline (speedup 1.0000x reference)
import functools

import jax
import jax.numpy as jnp
from jax import lax
from jax.experimental import pallas as pl
from jax.experimental.pallas import tpu as pltpu
from jax.experimental.pallas import tpu_sc as plsc

VOCAB = 1000000
DIM = 64
B = 16384
L = 200

NC = 2
NS = 16
NW = NC * NS

N = B * L
ROWS_PER_W = N // NW
IDX_MINOR = 128
CHUNK_IDX_ROWS = 8
CHUNK = IDX_MINOR * CHUNK_IDX_ROWS
NCHUNKS = ROWS_PER_W // CHUNK


def _sc_gather(h2d, table):
    mesh = plsc.VectorSubcoreMesh(core_axis_name="c", subcore_axis_name="s")

    @functools.partial(
        pl.kernel,
        out_type=jax.ShapeDtypeStruct((N, DIM), jnp.float32),
        mesh=mesh,
        scratch_types=[
            pltpu.VMEM((CHUNK_IDX_ROWS, IDX_MINOR), jnp.int32),
            pltpu.VMEM((CHUNK, DIM), jnp.float32),
            pltpu.SemaphoreType.DMA,
        ],
    )
    def k(h_hbm, table_hbm, out_hbm, idx_v, rows_v, sem):
        wid = lax.axis_index("s") * NC + lax.axis_index("c")
        base = wid * ROWS_PER_W

        @pl.loop(0, NCHUNKS)
        def _chunk(g):
            row0 = base + g * CHUNK
            pltpu.sync_copy(
                h_hbm.at[pl.ds((row0 // IDX_MINOR), CHUNK_IDX_ROWS)], idx_v)
            cps = [
                pltpu.async_copy(
                    table_hbm.at[idx_v.at[j]],
                    rows_v.at[pl.ds(j * IDX_MINOR, IDX_MINOR)],
                    sem,
                )
                for j in range(CHUNK_IDX_ROWS)
            ]
            for cp in cps:
                cp.wait()
            pltpu.sync_copy(rows_v, out_hbm.at[pl.ds(row0, CHUNK)])

    return k(h2d, table)


def kernel(h, table):
    h2d = h.reshape(N // IDX_MINOR, IDX_MINOR)
    out = _sc_gather(h2d, table)
    return out.reshape(B, L * DIM)


# speedup vs baseline: 1.5702x; 1.5702x over previous
import functools

import jax
import jax.numpy as jnp
from jax import lax
from jax.experimental import pallas as pl
from jax.experimental.pallas import tpu as pltpu
from jax.experimental.pallas import tpu_sc as plsc

VOCAB = 1000000
DIM = 64
B = 16384
L = 200

NC = 2
NS = 16
NW = NC * NS

N = B * L
ROWS_PER_W = N // NW
IDX_MINOR = 128
CHUNK_IDX_ROWS = 8
CHUNK = IDX_MINOR * CHUNK_IDX_ROWS
NCHUNKS = ROWS_PER_W // CHUNK


def _sc_gather(h2d, table):
    mesh = plsc.VectorSubcoreMesh(core_axis_name="c", subcore_axis_name="s")

    @functools.partial(
        pl.kernel,
        out_type=jax.ShapeDtypeStruct((N, DIM), jnp.float32),
        mesh=mesh,
        compiler_params=pltpu.CompilerParams(use_tc_tiling_on_sc=False),
        scratch_types=[
            pltpu.VMEM((CHUNK_IDX_ROWS, IDX_MINOR), jnp.int32),
            pltpu.VMEM((CHUNK, DIM), jnp.float32),
            pltpu.SemaphoreType.DMA,
        ],
    )
    def k(h_hbm, table_hbm, out_hbm, idx_v, rows_v, sem):
        wid = lax.axis_index("s") * NC + lax.axis_index("c")
        base = wid * ROWS_PER_W

        @pl.loop(0, NCHUNKS)
        def _chunk(g):
            row0 = pl.multiple_of(base + g * CHUNK, CHUNK)
            irow0 = pl.multiple_of(
                (base + g * CHUNK) // IDX_MINOR, CHUNK_IDX_ROWS)
            pltpu.sync_copy(
                h_hbm.at[pl.ds(irow0, CHUNK_IDX_ROWS)], idx_v)
            cps = [
                pltpu.async_copy(
                    table_hbm.at[idx_v.at[j]],
                    rows_v.at[pl.ds(j * IDX_MINOR, IDX_MINOR)],
                    sem,
                )
                for j in range(CHUNK_IDX_ROWS)
            ]
            for cp in cps:
                cp.wait()
            pltpu.sync_copy(rows_v, out_hbm.at[pl.ds(row0, CHUNK)])

    return k(h2d, table)


def kernel(h, table):
    h2d = h.reshape(N // IDX_MINOR, IDX_MINOR)
    out = _sc_gather(h2d, table)
    return out.reshape(B, L * DIM)


# speedup vs baseline: 1.6462x; 1.0484x over previous
import functools

import jax
import jax.numpy as jnp
from jax import lax
from jax.experimental import pallas as pl
from jax.experimental.pallas import tpu as pltpu
from jax.experimental.pallas import tpu_sc as plsc

VOCAB = 1000000
DIM = 64
B = 16384
L = 200

NC = 2
NS = 16
NW = NC * NS

N = B * L
ROWS_PER_W = N // NW
IDX_MINOR = 128
IDX_ROWS = 4
CHUNK = IDX_MINOR * IDX_ROWS
NCHUNKS = ROWS_PER_W // CHUNK


def _sc_gather(h2d, table):
    mesh = plsc.VectorSubcoreMesh(core_axis_name="c", subcore_axis_name="s")

    @functools.partial(
        pl.kernel,
        out_type=jax.ShapeDtypeStruct((N, DIM), jnp.float32),
        mesh=mesh,
        compiler_params=pltpu.CompilerParams(use_tc_tiling_on_sc=False),
        scratch_types=[
            pltpu.VMEM((IDX_ROWS, IDX_MINOR), jnp.int32),
            pltpu.VMEM((IDX_ROWS, IDX_MINOR), jnp.int32),
            pltpu.VMEM((CHUNK, DIM), jnp.float32),
            pltpu.VMEM((CHUNK, DIM), jnp.float32),
            pltpu.SemaphoreType.DMA,
            pltpu.SemaphoreType.DMA,
            pltpu.SemaphoreType.DMA,
            pltpu.SemaphoreType.DMA,
            pltpu.SemaphoreType.DMA,
            pltpu.SemaphoreType.DMA,
        ],
    )
    def k(h_hbm, table_hbm, out_hbm, idx0, idx1, rows0, rows1,
          si0, si1, sg0, sg1, so0, so1):
        idx = (idx0, idx1)
        rows = (rows0, rows1)
        si = (si0, si1)
        sg = (sg0, sg1)
        so = (so0, so1)

        wid = lax.axis_index("s") * NC + lax.axis_index("c")
        base = wid * ROWS_PER_W

        def idx_cp(c, b, sem):
            irow0 = pl.multiple_of((base + c * CHUNK) // IDX_MINOR, IDX_ROWS)
            return pltpu.make_async_copy(
                h_hbm.at[pl.ds(irow0, IDX_ROWS)], idx[b], sem)

        def gather_cps(b, sem):
            return [
                pltpu.make_async_copy(
                    table_hbm.at[idx[b].at[j]],
                    rows[b].at[pl.ds(j * IDX_MINOR, IDX_MINOR)],
                    sem,
                )
                for j in range(IDX_ROWS)
            ]

        def out_cp(c, b, sem):
            row0 = pl.multiple_of(base + c * CHUNK, CHUNK)
            return pltpu.make_async_copy(
                rows[b], out_hbm.at[pl.ds(row0, CHUNK)], sem)

        idx_cp(0, 0, si[0]).start()
        idx_cp(1, 1, si[1]).start()
        idx_cp(0, 0, si[0]).wait()
        for cp in gather_cps(0, sg[0]):
            cp.start()

        @pl.loop(0, NCHUNKS, step=2)
        def _pair(g):
            for b in (0, 1):
                c = g + b
                o = 1 - b
                @pl.when(c + 1 < NCHUNKS)
                def _():
                    idx_cp(c + 1, o, si[o]).wait()

                    @pl.when(c >= 1)
                    def _():
                        out_cp(c - 1, o, so[o]).wait()

                    for cp in gather_cps(o, sg[o]):
                        cp.start()

                for cp in gather_cps(b, sg[b]):
                    cp.wait()
                out_cp(c, b, so[b]).start()

                @pl.when(c + 2 < NCHUNKS)
                def _():
                    idx_cp(c + 2, b, si[b]).start()

        out_cp(NCHUNKS - 2, 0, so[0]).wait()
        out_cp(NCHUNKS - 1, 1, so[1]).wait()

    return k(h2d, table)


def kernel(h, table):
    h2d = h.reshape(N // IDX_MINOR, IDX_MINOR)
    out = _sc_gather(h2d, table)
    return out.reshape(B, L * DIM)


# speedup vs baseline: 1.6462x; 1.0000x over previous
import functools

import jax
import jax.numpy as jnp
from jax import lax
from jax.experimental import pallas as pl
from jax.experimental.pallas import tpu as pltpu
from jax.experimental.pallas import tpu_sc as plsc

VOCAB = 1000000
DIM = 64
B = 16384
L = 200

NC = 2
NS = 16
NW = NC * NS

N = B * L
ROWS_PER_W = N // NW
IDX_MINOR = 128
IDX_ROWS = 2
CHUNK = IDX_MINOR * IDX_ROWS
NBUF = 4
NCHUNKS = ROWS_PER_W // CHUNK


def _sc_gather(h2d, table):
    mesh = plsc.VectorSubcoreMesh(core_axis_name="c", subcore_axis_name="s")

    @functools.partial(
        pl.kernel,
        out_type=jax.ShapeDtypeStruct((N, DIM), jnp.float32),
        mesh=mesh,
        compiler_params=pltpu.CompilerParams(use_tc_tiling_on_sc=False),
        scratch_types=(
            [pltpu.VMEM((IDX_ROWS, IDX_MINOR), jnp.int32)] * NBUF
            + [pltpu.VMEM((CHUNK, DIM), jnp.float32)] * NBUF
            + [pltpu.SemaphoreType.DMA] * (3 * NBUF)
        ),
    )
    def k(h_hbm, table_hbm, out_hbm, *bufs):
        idx = bufs[:NBUF]
        rows = bufs[NBUF:2 * NBUF]
        si = bufs[2 * NBUF:2 * NBUF + NBUF]
        sg = bufs[3 * NBUF:3 * NBUF + NBUF]
        so = bufs[4 * NBUF:]

        wid = lax.axis_index("s") * NC + lax.axis_index("c")
        base = wid * ROWS_PER_W

        def idx_cp(c, b):
            irow0 = pl.multiple_of((base + c * CHUNK) // IDX_MINOR, IDX_ROWS)
            return pltpu.make_async_copy(
                h_hbm.at[pl.ds(irow0, IDX_ROWS)], idx[b], si[b])

        def gather_cps(b):
            return [
                pltpu.make_async_copy(
                    table_hbm.at[idx[b].at[j]],
                    rows[b].at[pl.ds(j * IDX_MINOR, IDX_MINOR)],
                    sg[b],
                )
                for j in range(IDX_ROWS)
            ]

        def out_cp(c, b):
            row0 = pl.multiple_of(base + c * CHUNK, CHUNK)
            return pltpu.make_async_copy(
                rows[b], out_hbm.at[pl.ds(row0, CHUNK)], so[b])

        for b in range(NBUF):
            idx_cp(b, b).start()
        for b in range(NBUF - 1):
            idx_cp(b, b).wait()
            for cp in gather_cps(b):
                cp.start()

        @pl.loop(0, NCHUNKS, step=NBUF)
        def _group(g):
            for b in range(NBUF):
                c = g + b
                a = (b + NBUF - 1) % NBUF

                @pl.when(c + NBUF - 1 < NCHUNKS)
                def _():
                    idx_cp(c + NBUF - 1, a).wait()

                    @pl.when(c >= 1)
                    def _():
                        out_cp(c - 1, a).wait()

                    for cp in gather_cps(a):
                        cp.start()

                for cp in gather_cps(b):
                    cp.wait()
                out_cp(c, b).start()

                @pl.when(c + NBUF < NCHUNKS)
                def _():
                    idx_cp(c + NBUF, b).start()

        for d in range(NBUF, 0, -1):
            c = NCHUNKS - d
            out_cp(c, c % NBUF).wait()

    return k(h2d, table)


def kernel(h, table):
    h2d = h.reshape(N // IDX_MINOR, IDX_MINOR)
    out = _sc_gather(h2d, table)
    return out.reshape(B, L * DIM)
